# baseline (speedup 1.0000x reference)
import functools

import jax
import jax.numpy as jnp
from jax import lax
from jax.experimental import pallas as pl
from jax.experimental.pallas import tpu as pltpu
from jax.experimental.pallas import tpu_sc as plsc

VOCAB = 100000
HIDDEN = 768
EPS = 1e-07
L = 16
NCHUNK = HIDDEN // L
NC, NS = 2, 16
NW = NC * NS
N_IDS = 4 * 4096
BPW = N_IDS // NW
C = 32
G = BPW // C
K = G // 2
REL_PAD = 512
REL_PW = REL_PAD // NW
NACC = 8


def _perm16(v, idx):
    dn = lax.GatherDimensionNumbers(
        offset_dims=(), collapsed_slice_dims=(0,), start_index_map=(0,))
    return lax.gather(v, idx[:, None], dn, slice_sizes=(1,),
                      mode=lax.GatherScatterMode.PROMISE_IN_BOUNDS)


def _xlane_sum(v):
    base = lax.iota(jnp.int32, L)
    for m in (1, 2, 4, 8):
        v = v + _perm16(v, base ^ m)
    return v


def _rsqrt16(x):
    i = lax.bitcast_convert_type(x, jnp.int32)
    i = jnp.int32(0x5F3759DF) - lax.shift_right_logical(i, 1)
    y = lax.bitcast_convert_type(i, jnp.float32)
    for _ in range(3):
        y = y * (1.5 - 0.5 * x * y * y)
    return y


def _ln_row(src, dst, r, gam_v, bet_v):
    sums = [jnp.zeros((L,), jnp.float32) for _ in range(NACC)]
    sqs = [jnp.zeros((L,), jnp.float32) for _ in range(NACC)]
    for j in range(NCHUNK):
        a = j % NACC
        x = src[r, pl.ds(j * L, L)]
        sums[a] = sums[a] + x
        sqs[a] = sqs[a] + x * x
    while len(sums) > 1:
        sums = [sums[i] + sums[i + 1] for i in range(0, len(sums), 2)]
        sqs = [sqs[i] + sqs[i + 1] for i in range(0, len(sqs), 2)]
    mv = _xlane_sum(sums[0]) * (1.0 / HIDDEN)
    var = _xlane_sum(sqs[0]) * (1.0 / HIDDEN) - mv * mv
    rstd = _rsqrt16(var + EPS)
    for j in range(NCHUNK):
        x = src[r, pl.ds(j * L, L)]
        y = (x - mv) * rstd
        if gam_v is not None:
            y = y * gam_v[pl.ds(j * L, L)] + bet_v[pl.ds(j * L, L)]
        dst[r, pl.ds(j * L, L)] = y


def _make_sc_kernel():
    mesh = plsc.VectorSubcoreMesh(core_axis_name="c", subcore_axis_name="s")

    @functools.partial(
        pl.kernel,
        mesh=mesh,
        out_type=[
            jax.ShapeDtypeStruct((N_IDS, HIDDEN), jnp.float32),
            jax.ShapeDtypeStruct((REL_PAD, HIDDEN), jnp.float32),
        ],
        scratch_types=[
            pltpu.VMEM((C,), jnp.int32),
            pltpu.VMEM((C,), jnp.int32),
            pltpu.VMEM((C, HIDDEN), jnp.float32),
            pltpu.VMEM((C, HIDDEN), jnp.float32),
            pltpu.VMEM((C, HIDDEN), jnp.float32),
            pltpu.VMEM((C, HIDDEN), jnp.float32),
            pltpu.VMEM((REL_PW, HIDDEN), jnp.float32),
            pltpu.VMEM((HIDDEN,), jnp.float32),
            pltpu.VMEM((HIDDEN,), jnp.float32),
            pltpu.SemaphoreType.DMA,
            pltpu.SemaphoreType.DMA,
            pltpu.SemaphoreType.DMA,
            pltpu.SemaphoreType.DMA,
        ],
    )
    def sc_kernel(ids_hbm, table_hbm, rel_hbm, gam_hbm, bet_hbm,
                  word_out_hbm, rel_out_hbm,
                  idx0, idx1, in0, in1, out0, out1, rel_v, gam_v, bet_v,
                  gsem0, gsem1, wsem0, wsem1):
        wid = lax.axis_index("s") * NC + lax.axis_index("c")
        base0 = wid * BPW

        def ln_block(src, dst, n, gam, bet):
            @plsc.parallel_loop(0, n, 1, unroll=8)
            def row_body(r):
                _ln_row(src, dst, r, gam, bet)

        def start_gather(idx_v, in_v, sem, g):
            pltpu.sync_copy(ids_hbm.at[pl.ds(base0 + g * C, C)], idx_v)
            pltpu.async_copy(table_hbm.at[idx_v], in_v, sem)

        def wait_gather(idx_v, in_v, sem):
            pltpu.make_async_copy(table_hbm.at[idx_v], in_v, sem).wait()

        def start_wb(out_v, sem, g):
            pltpu.async_copy(out_v, word_out_hbm.at[pl.ds(base0 + g * C, C)],
                             sem)

        def wait_wb(out_v, sem, g):
            pltpu.make_async_copy(
                out_v, word_out_hbm.at[pl.ds(base0 + g * C, C)], sem).wait()

        start_gather(idx0, in0, gsem0, 0)
        start_gather(idx1, in1, gsem1, 1)

        rbase = wid * REL_PW
        pltpu.sync_copy(gam_hbm, gam_v)
        pltpu.sync_copy(bet_hbm, bet_v)
        pltpu.sync_copy(rel_hbm.at[pl.ds(rbase, REL_PW)], rel_v)
        ln_block(rel_v, rel_v, REL_PW, gam_v, bet_v)
        pltpu.sync_copy(rel_v, rel_out_hbm.at[pl.ds(rbase, REL_PW)])

        def pipe_body(k, carry):
            g0 = k * 2

            wait_gather(idx0, in0, gsem0)

            @pl.when(k > 0)
            def _():
                wait_wb(out0, wsem0, g0 - 2)

            ln_block(in0, out0, C, None, None)
            start_wb(out0, wsem0, g0)

            @pl.when(k < K - 1)
            def _():
                start_gather(idx0, in0, gsem0, g0 + 2)

            wait_gather(idx1, in1, gsem1)

            @pl.when(k > 0)
            def _():
                wait_wb(out1, wsem1, g0 - 1)

            ln_block(in1, out1, C, None, None)
            start_wb(out1, wsem1, g0 + 1)

            @pl.when(k < K - 1)
            def _():
                start_gather(idx1, in1, gsem1, g0 + 3)

            return carry

        lax.fori_loop(0, K, pipe_body, 0)
        wait_wb(out0, wsem0, G - 2)
        wait_wb(out1, wsem1, G - 1)

    return sc_kernel


_SC_KERNEL = _make_sc_kernel()


def kernel(input_ids, word_table, relative_embedding, rel_ln_gamma, rel_ln_beta):
    ids = input_ids.reshape(-1).astype(jnp.int32)
    rel_padded = jnp.pad(relative_embedding, ((0, REL_PAD - relative_embedding.shape[0]), (0, 0)))
    word_flat, rel_out = _SC_KERNEL(ids, word_table, rel_padded,
                                    rel_ln_gamma, rel_ln_beta)
    word = word_flat.reshape(input_ids.shape + (HIDDEN,))
    return (word, rel_out[: relative_embedding.shape[0]])


# speedup vs baseline: 1.1794x; 1.1794x over previous
import functools

import jax
import jax.numpy as jnp
from jax import lax
from jax.experimental import pallas as pl
from jax.experimental.pallas import tpu as pltpu
from jax.experimental.pallas import tpu_sc as plsc

VOCAB = 100000
HIDDEN = 768
EPS = 1e-07
L = 16
NCHUNK = HIDDEN // L
NC, NS = 2, 16
NW = NC * NS
N_IDS = 4 * 4096
BPW = N_IDS // NW
C = 32
G = BPW // C
K = G // 2
REL_PAD = 512
REL_PW = REL_PAD // NW
NACC = 4


def _perm16(v, idx):
    dn = lax.GatherDimensionNumbers(
        offset_dims=(), collapsed_slice_dims=(0,), start_index_map=(0,))
    return lax.gather(v, idx[:, None], dn, slice_sizes=(1,),
                      mode=lax.GatherScatterMode.PROMISE_IN_BOUNDS)


def _xlane_sum(v):
    base = lax.iota(jnp.int32, L)
    for m in (1, 2, 4, 8):
        v = v + _perm16(v, base ^ m)
    return v


def _rsqrt16(x):
    i = lax.bitcast_convert_type(x, jnp.int32)
    i = jnp.int32(0x5F3759DF) - lax.shift_right_logical(i, 1)
    y = lax.bitcast_convert_type(i, jnp.float32)
    for _ in range(3):
        y = y * (1.5 - 0.5 * x * y * y)
    return y


def _ln_row(src, dst, r, gam_v, bet_v):
    sums = [jnp.zeros((L,), jnp.float32) for _ in range(NACC)]
    sqs = [jnp.zeros((L,), jnp.float32) for _ in range(NACC)]
    for j in range(NCHUNK):
        a = j % NACC
        x = src[r, pl.ds(j * L, L)]
        sums[a] = sums[a] + x
        sqs[a] = sqs[a] + x * x
    while len(sums) > 1:
        sums = [sums[i] + sums[i + 1] for i in range(0, len(sums), 2)]
        sqs = [sqs[i] + sqs[i + 1] for i in range(0, len(sqs), 2)]
    mv = _xlane_sum(sums[0]) * (1.0 / HIDDEN)
    var = _xlane_sum(sqs[0]) * (1.0 / HIDDEN) - mv * mv
    rstd = _rsqrt16(var + EPS)
    for j in range(NCHUNK):
        x = src[r, pl.ds(j * L, L)]
        y = (x - mv) * rstd
        if gam_v is not None:
            y = y * gam_v[pl.ds(j * L, L)] + bet_v[pl.ds(j * L, L)]
        dst[r, pl.ds(j * L, L)] = y


def _make_sc_kernel():
    mesh = plsc.VectorSubcoreMesh(core_axis_name="c", subcore_axis_name="s")

    @functools.partial(
        pl.kernel,
        mesh=mesh,
        out_type=[
            jax.ShapeDtypeStruct((N_IDS, HIDDEN), jnp.float32),
            jax.ShapeDtypeStruct((REL_PAD, HIDDEN), jnp.float32),
        ],
        scratch_types=[
            pltpu.VMEM((C,), jnp.int32),
            pltpu.VMEM((C,), jnp.int32),
            pltpu.VMEM((C, HIDDEN), jnp.float32),
            pltpu.VMEM((C, HIDDEN), jnp.float32),
            pltpu.VMEM((C, HIDDEN), jnp.float32),
            pltpu.VMEM((C, HIDDEN), jnp.float32),
            pltpu.VMEM((REL_PW, HIDDEN), jnp.float32),
            pltpu.VMEM((HIDDEN,), jnp.float32),
            pltpu.VMEM((HIDDEN,), jnp.float32),
            pltpu.SemaphoreType.DMA,
            pltpu.SemaphoreType.DMA,
            pltpu.SemaphoreType.DMA,
            pltpu.SemaphoreType.DMA,
        ],
    )
    def sc_kernel(ids_hbm, table_hbm, rel_hbm, gam_hbm, bet_hbm,
                  word_out_hbm, rel_out_hbm,
                  idx0, idx1, in0, in1, out0, out1, rel_v, gam_v, bet_v,
                  gsem0, gsem1, wsem0, wsem1):
        wid = lax.axis_index("s") * NC + lax.axis_index("c")
        base0 = wid * BPW

        def ln_block(src, dst, n, gam, bet):
            @plsc.parallel_loop(0, n, 1, unroll=4)
            def row_body(r):
                _ln_row(src, dst, r, gam, bet)

        def start_gather(idx_v, in_v, sem, g):
            pltpu.sync_copy(ids_hbm.at[pl.ds(base0 + g * C, C)], idx_v)
            pltpu.async_copy(table_hbm.at[idx_v], in_v, sem)

        def wait_gather(idx_v, in_v, sem):
            pltpu.make_async_copy(table_hbm.at[idx_v], in_v, sem).wait()

        def start_wb(out_v, sem, g):
            pltpu.async_copy(out_v, word_out_hbm.at[pl.ds(base0 + g * C, C)],
                             sem)

        def wait_wb(out_v, sem, g):
            pltpu.make_async_copy(
                out_v, word_out_hbm.at[pl.ds(base0 + g * C, C)], sem).wait()

        start_gather(idx0, in0, gsem0, 0)
        start_gather(idx1, in1, gsem1, 1)

        rbase = wid * REL_PW
        pltpu.sync_copy(gam_hbm, gam_v)
        pltpu.sync_copy(bet_hbm, bet_v)
        pltpu.sync_copy(rel_hbm.at[pl.ds(rbase, REL_PW)], rel_v)
        ln_block(rel_v, rel_v, REL_PW, gam_v, bet_v)
        pltpu.sync_copy(rel_v, rel_out_hbm.at[pl.ds(rbase, REL_PW)])

        def pipe_body(k, carry):
            g0 = k * 2

            wait_gather(idx0, in0, gsem0)

            @pl.when(k > 0)
            def _():
                wait_wb(out0, wsem0, g0 - 2)

            ln_block(in0, out0, C, None, None)
            start_wb(out0, wsem0, g0)

            @pl.when(k < K - 1)
            def _():
                start_gather(idx0, in0, gsem0, g0 + 2)

            wait_gather(idx1, in1, gsem1)

            @pl.when(k > 0)
            def _():
                wait_wb(out1, wsem1, g0 - 1)

            ln_block(in1, out1, C, None, None)
            start_wb(out1, wsem1, g0 + 1)

            @pl.when(k < K - 1)
            def _():
                start_gather(idx1, in1, gsem1, g0 + 3)

            return carry

        lax.fori_loop(0, K, pipe_body, 0)
        wait_wb(out0, wsem0, G - 2)
        wait_wb(out1, wsem1, G - 1)

    return sc_kernel


_SC_KERNEL = _make_sc_kernel()


def kernel(input_ids, word_table, relative_embedding, rel_ln_gamma, rel_ln_beta):
    ids = input_ids.reshape(-1).astype(jnp.int32)
    rel_padded = jnp.pad(relative_embedding, ((0, REL_PAD - relative_embedding.shape[0]), (0, 0)))
    word_flat, rel_out = _SC_KERNEL(ids, word_table, rel_padded,
                                    rel_ln_gamma, rel_ln_beta)
    word = word_flat.reshape(input_ids.shape + (HIDDEN,))
    return (word, rel_out[: relative_embedding.shape[0]])


# speedup vs baseline: 1.3100x; 1.1107x over previous
import functools

import jax
import jax.numpy as jnp
from jax import lax
from jax.experimental import pallas as pl
from jax.experimental.pallas import tpu as pltpu
from jax.experimental.pallas import tpu_sc as plsc

VOCAB = 100000
HIDDEN = 768
EPS = 1e-07
L = 16
NCHUNK = HIDDEN // L
NC, NS = 2, 16
NW = NC * NS
N_IDS = 4 * 4096
BPW = N_IDS // NW
C = 32
G = BPW // C
K = G // 2
REL_PAD = 512
REL_PW = REL_PAD // NW
NACC = 4


def _perm16(v, idx):
    dn = lax.GatherDimensionNumbers(
        offset_dims=(), collapsed_slice_dims=(0,), start_index_map=(0,))
    return lax.gather(v, idx[:, None], dn, slice_sizes=(1,),
                      mode=lax.GatherScatterMode.PROMISE_IN_BOUNDS)


def _xlane_sum(v):
    base = lax.iota(jnp.int32, L)
    for m in (1, 2, 4, 8):
        v = v + _perm16(v, base ^ m)
    return v


def _rsqrt16(x):
    i = lax.bitcast_convert_type(x, jnp.int32)
    i = jnp.int32(0x5F3759DF) - lax.shift_right_logical(i, 1)
    y = lax.bitcast_convert_type(i, jnp.float32)
    for _ in range(3):
        y = y * (1.5 - 0.5 * x * y * y)
    return y


def _ln_row(src, dst, r, gam_v, bet_v):
    sums = [jnp.zeros((L,), jnp.float32) for _ in range(NACC)]
    sqs = [jnp.zeros((L,), jnp.float32) for _ in range(NACC)]
    for j in range(NCHUNK):
        a = j % NACC
        x = src[r, pl.ds(j * L, L)]
        sums[a] = sums[a] + x
        sqs[a] = sqs[a] + x * x
    while len(sums) > 1:
        sums = [sums[i] + sums[i + 1] for i in range(0, len(sums), 2)]
        sqs = [sqs[i] + sqs[i + 1] for i in range(0, len(sqs), 2)]
    mv = _xlane_sum(sums[0]) * (1.0 / HIDDEN)
    var = _xlane_sum(sqs[0]) * (1.0 / HIDDEN) - mv * mv
    rstd = _rsqrt16(var + EPS)
    for j in range(NCHUNK):
        x = src[r, pl.ds(j * L, L)]
        y = (x - mv) * rstd
        if gam_v is not None:
            y = y * gam_v[pl.ds(j * L, L)] + bet_v[pl.ds(j * L, L)]
        dst[r, pl.ds(j * L, L)] = y


def _make_sc_kernel():
    mesh = plsc.VectorSubcoreMesh(core_axis_name="c", subcore_axis_name="s")

    @functools.partial(
        pl.kernel,
        mesh=mesh,
        out_type=[
            jax.ShapeDtypeStruct((N_IDS, HIDDEN), jnp.float32),
            jax.ShapeDtypeStruct((REL_PAD, HIDDEN), jnp.float32),
        ],
        scratch_types=[
            pltpu.VMEM((BPW,), jnp.int32),
            pltpu.VMEM((C, HIDDEN), jnp.float32),
            pltpu.VMEM((C, HIDDEN), jnp.float32),
            pltpu.VMEM((C, HIDDEN), jnp.float32),
            pltpu.VMEM((C, HIDDEN), jnp.float32),
            pltpu.VMEM((REL_PW, HIDDEN), jnp.float32),
            pltpu.VMEM((HIDDEN,), jnp.float32),
            pltpu.VMEM((HIDDEN,), jnp.float32),
            pltpu.SemaphoreType.DMA,
            pltpu.SemaphoreType.DMA,
            pltpu.SemaphoreType.DMA,
            pltpu.SemaphoreType.DMA,
        ],
    )
    def sc_kernel(ids_hbm, table_hbm, rel_hbm, gam_hbm, bet_hbm,
                  word_out_hbm, rel_out_hbm,
                  idx_all, in0, in1, out0, out1, rel_v, gam_v, bet_v,
                  gsem0, gsem1, wsem0, wsem1):
        wid = lax.axis_index("s") * NC + lax.axis_index("c")
        base0 = wid * BPW

        def ln_block(src, dst, n, gam, bet):
            @plsc.parallel_loop(0, n, 1, unroll=4)
            def row_body(r):
                _ln_row(src, dst, r, gam, bet)

        def start_gather(in_v, sem, g):
            pltpu.async_copy(table_hbm.at[idx_all.at[pl.ds(g * C, C)]],
                             in_v, sem)

        def wait_gather(in_v, sem, g):
            pltpu.make_async_copy(table_hbm.at[idx_all.at[pl.ds(g * C, C)]],
                                  in_v, sem).wait()

        def start_wb(out_v, sem, g):
            pltpu.async_copy(out_v, word_out_hbm.at[pl.ds(base0 + g * C, C)],
                             sem)

        def wait_wb(out_v, sem, g):
            pltpu.make_async_copy(
                out_v, word_out_hbm.at[pl.ds(base0 + g * C, C)], sem).wait()

        pltpu.sync_copy(ids_hbm.at[pl.ds(base0, BPW)], idx_all)
        start_gather(in0, gsem0, 0)
        start_gather(in1, gsem1, 1)

        rbase = wid * REL_PW
        pltpu.sync_copy(gam_hbm, gam_v)
        pltpu.sync_copy(bet_hbm, bet_v)
        pltpu.sync_copy(rel_hbm.at[pl.ds(rbase, REL_PW)], rel_v)
        ln_block(rel_v, rel_v, REL_PW, gam_v, bet_v)
        pltpu.sync_copy(rel_v, rel_out_hbm.at[pl.ds(rbase, REL_PW)])

        def pipe_body(k, carry):
            g0 = k * 2

            wait_gather(in0, gsem0, g0)

            @pl.when(k > 0)
            def _():
                wait_wb(out0, wsem0, g0 - 2)

            ln_block(in0, out0, C, None, None)
            start_wb(out0, wsem0, g0)

            @pl.when(k < K - 1)
            def _():
                start_gather(in0, gsem0, g0 + 2)

            wait_gather(in1, gsem1, g0 + 1)

            @pl.when(k > 0)
            def _():
                wait_wb(out1, wsem1, g0 - 1)

            ln_block(in1, out1, C, None, None)
            start_wb(out1, wsem1, g0 + 1)

            @pl.when(k < K - 1)
            def _():
                start_gather(in1, gsem1, g0 + 3)

            return carry

        lax.fori_loop(0, K, pipe_body, 0)
        wait_wb(out0, wsem0, G - 2)
        wait_wb(out1, wsem1, G - 1)

    return sc_kernel


_SC_KERNEL = _make_sc_kernel()


def kernel(input_ids, word_table, relative_embedding, rel_ln_gamma, rel_ln_beta):
    ids = input_ids.reshape(-1).astype(jnp.int32)
    rel_padded = jnp.pad(relative_embedding, ((0, REL_PAD - relative_embedding.shape[0]), (0, 0)))
    word_flat, rel_out = _SC_KERNEL(ids, word_table, rel_padded,
                                    rel_ln_gamma, rel_ln_beta)
    word = word_flat.reshape(input_ids.shape + (HIDDEN,))
    return (word, rel_out[: relative_embedding.shape[0]])
